# baseline (speedup 1.0000x reference)
import functools

import jax
import jax.numpy as jnp
from jax import lax
from jax.experimental import pallas as pl
from jax.experimental.pallas import tpu as pltpu
from jax.experimental.pallas import tpu_sc as plsc

_EPS = 1e-5
_NC = 2
_NS = 16
_NW = _NC * _NS
_K = 80


def _sc_mesh():
    return plsc.VectorSubcoreMesh(core_axis_name="c", subcore_axis_name="s")


def _sc_degree(dst3, ones_blk, zeros_blk, n):
    j = dst3.shape[1]
    d = ones_blk.shape[1]
    rows = n // _NS

    @functools.partial(
        pl.kernel,
        mesh=_sc_mesh(),
        out_type=jax.ShapeDtypeStruct((_NC, _NS, rows, d), jnp.float32),
        scratch_types=[
            pltpu.VMEM((j, _K), jnp.int32),
            pltpu.VMEM((_K, d), jnp.float32),
            pltpu.VMEM_SHARED((n, d), jnp.float32),
        ],
    )
    def k(dst_hbm, ones_hbm, z_hbm, out_hbm, dst_v, ones_v, acc_sh):
        c = lax.axis_index("c")
        s = lax.axis_index("s")
        wid = c * _NS + s
        pltpu.sync_copy(z_hbm, acc_sh.at[pl.ds(s * rows, rows)])
        pltpu.sync_copy(dst_hbm.at[wid], dst_v)
        pltpu.sync_copy(ones_hbm, ones_v)
        plsc.subcore_barrier()

        def body(i, carry):
            pltpu.sync_copy(ones_v, acc_sh.at[dst_v.at[i]], add=True)
            return carry

        lax.fori_loop(0, j, body, 0)
        plsc.subcore_barrier()
        pltpu.sync_copy(acc_sh.at[pl.ds(s * rows, rows)], out_hbm.at[c, s])

    return k(dst3, ones_blk, zeros_blk).reshape(_NC, n, d)


def _sc_scatter(table, src3, dst3, zeros_blk):
    n, d = table.shape
    j = src3.shape[1]
    ph = j // 2
    n_acc = zeros_blk.shape[0] * _NS
    rows = n_acc // _NS

    @functools.partial(
        pl.kernel,
        mesh=_sc_mesh(),
        out_type=jax.ShapeDtypeStruct((_NC, _NS, rows, d), jnp.float32),
        scratch_types=[
            pltpu.VMEM((j, 128), jnp.int32),
            pltpu.VMEM((j, 128), jnp.int32),
            pltpu.VMEM((128, d), jnp.float32),
            pltpu.VMEM_SHARED((n_acc, d), jnp.float32),
            pltpu.SemaphoreType.DMA,
        ],
    )
    def k(tab_hbm, src_hbm, dst_hbm, z_hbm, out_hbm,
          src_v, dst_v, rows_v, acc_sh, sem):
        c = lax.axis_index("c")
        s = lax.axis_index("s")
        wid = c * _NS + s
        pltpu.sync_copy(src_hbm.at[wid], src_v)
        pltpu.sync_copy(dst_hbm.at[wid], dst_v)
        pltpu.sync_copy(z_hbm, acc_sh.at[pl.ds(s * rows, rows)])
        plsc.subcore_barrier()

        def body(i, carry):
            pltpu.async_copy(tab_hbm.at[src_v.at[i]], rows_v, sem).wait()
            pltpu.sync_copy(rows_v, acc_sh.at[dst_v.at[i]], add=True)
            return carry

        lax.fori_loop(0, j, body, 0)
        plsc.subcore_barrier()
        pltpu.sync_copy(acc_sh.at[pl.ds(s * rows, rows)], out_hbm.at[c, s])

    return k(table, src3, dst3, zeros_blk).reshape(_NC, n_acc, d)


def _dinv(deg_ref):
    deg = deg_ref[0, :, 0:1] + deg_ref[1, :, 0:1] + 1.0
    return lax.rsqrt(deg)


def _tc_scale_mm(x, w, degp):
    def body(x_ref, w_ref, deg_ref, o_ref):
        o_ref[...] = jnp.dot(x_ref[...], w_ref[...],
                             preferred_element_type=jnp.float32) * _dinv(deg_ref)

    return pl.pallas_call(
        body,
        out_shape=jax.ShapeDtypeStruct((x.shape[0], w.shape[1]), jnp.float32),
    )(x, w, degp)


def _tc_mid(accp, g1, degp, b1, gamma, beta, w2):
    n = g1.shape[0]

    def body(acc_ref, g_ref, deg_ref, b_ref, ga_ref, be_ref, w_ref, o_ref):
        dinv = _dinv(deg_ref)
        m = (acc_ref[0] + acc_ref[1] + g_ref[...]) * dinv + b_ref[...]
        mean = jnp.mean(m, axis=0, keepdims=True)
        cen = m - mean
        var = jnp.mean(cen * cen, axis=0, keepdims=True)
        xn = cen * lax.rsqrt(var + _EPS) * ga_ref[...] + be_ref[...]
        xr = jnp.maximum(xn, 0.0)
        o_ref[...] = jnp.dot(xr, w_ref[...],
                             preferred_element_type=jnp.float32) * dinv

    return pl.pallas_call(
        body,
        out_shape=jax.ShapeDtypeStruct((n, w2.shape[1]), jnp.float32),
    )(accp, g1, degp, b1, gamma, beta, w2)


def _tc_final(accp, g2, degp, b2, d_out):
    n = g2.shape[0]

    def body(acc_ref, g_ref, deg_ref, b_ref, o_ref):
        acc = acc_ref[0, :, :d_out] + acc_ref[1, :, :d_out] + g_ref[:, :d_out]
        o_ref[...] = acc * _dinv(deg_ref) + b_ref[...]

    return pl.pallas_call(
        body,
        out_shape=jax.ShapeDtypeStruct((n, d_out), jnp.float32),
    )(accp, g2, degp, b2)


def kernel(x, edge_index, W1, b1, gamma, beta, W2, b2):
    n, d_in = x.shape
    e = edge_index.shape[1]
    d_hid = W1.shape[1]
    d_out = W2.shape[1]
    j = e // (_NW * _K)
    rows = n // _NS

    ept = e // _NW
    j2 = 80
    pad = j2 * 128 - ept
    srcd = jnp.pad(edge_index[0].reshape(_NW, ept),
                   ((0, 0), (0, pad))).reshape(_NW, j2, 128)
    junk = (n + jnp.arange(_NW, dtype=edge_index.dtype) % _NS)[:, None]
    dstd = jnp.concatenate(
        [edge_index[1].reshape(_NW, ept),
         jnp.broadcast_to(junk, (_NW, pad))], axis=1).reshape(_NW, j2, 128)
    src3 = edge_index[0].reshape(_NW, j, _K)
    dst3 = edge_index[1].reshape(_NW, j, _K)
    ones_blk = jnp.ones((_K, d_hid), jnp.float32)
    zh = jnp.zeros((rows, d_hid), jnp.float32)
    zacc = jnp.zeros((rows + 1, d_hid), jnp.float32)

    w2p = jnp.pad(W2, ((0, 0), (0, d_hid - d_out)))

    degp = _sc_degree(dst3, ones_blk, zh, n)[:, :, :16]
    g1 = _tc_scale_mm(x, W1, degp)
    acc1 = _sc_scatter(g1, srcd, dstd, zacc)[:, :n, :]
    g2 = _tc_mid(acc1, g1, degp, b1.reshape(1, -1), gamma.reshape(1, -1),
                 beta.reshape(1, -1), w2p)
    acc2 = _sc_scatter(g2, srcd, dstd, zacc)[:, :n, :]
    return _tc_final(acc2, g2, degp, b2.reshape(1, -1), d_out)


# speedup vs baseline: 2.6392x; 2.6392x over previous
import functools

import jax
import jax.numpy as jnp
from jax import lax
from jax.experimental import pallas as pl
from jax.experimental.pallas import tpu as pltpu
from jax.experimental.pallas import tpu_sc as plsc

_EPS = 1e-5
_NC = 2
_NS = 16
_NW = _NC * _NS
_K = 80


def _sc_mesh():
    return plsc.VectorSubcoreMesh(core_axis_name="c", subcore_axis_name="s")


def _sc_degree(dst3, ones_blk, zeros_blk, n):
    j = dst3.shape[1]
    d = ones_blk.shape[1]
    rows = n // _NS

    @functools.partial(
        pl.kernel,
        mesh=_sc_mesh(),
        out_type=jax.ShapeDtypeStruct((_NC, _NS, rows, d), jnp.float32),
        scratch_types=[
            pltpu.VMEM((j, _K), jnp.int32),
            pltpu.VMEM((_K, d), jnp.float32),
            pltpu.VMEM_SHARED((n, d), jnp.float32),
        ],
    )
    def k(dst_hbm, ones_hbm, z_hbm, out_hbm, dst_v, ones_v, acc_sh):
        c = lax.axis_index("c")
        s = lax.axis_index("s")
        wid = c * _NS + s
        pltpu.sync_copy(z_hbm, acc_sh.at[pl.ds(s * rows, rows)])
        pltpu.sync_copy(dst_hbm.at[wid], dst_v)
        pltpu.sync_copy(ones_hbm, ones_v)
        plsc.subcore_barrier()

        def body(i, carry):
            pltpu.sync_copy(ones_v, acc_sh.at[dst_v.at[i]], add=True)
            return carry

        lax.fori_loop(0, j, body, 0)
        plsc.subcore_barrier()
        pltpu.sync_copy(acc_sh.at[pl.ds(s * rows, rows)], out_hbm.at[c, s])

    return k(dst3, ones_blk, zeros_blk).reshape(_NC, n, d)


def _sc_scatter(table, src3, dst3, zeros_blk):
    n, d = table.shape
    j = src3.shape[1]
    kk = src3.shape[2]
    ph0 = 64
    ph1 = j - ph0
    rows = n // _NS

    @functools.partial(
        pl.kernel,
        mesh=_sc_mesh(),
        out_type=jax.ShapeDtypeStruct((_NC, _NS, rows, d), jnp.float32),
        scratch_types=[
            pltpu.VMEM((ph0, kk), jnp.int32),
            pltpu.VMEM((ph0, kk), jnp.int32),
            pltpu.VMEM((2, kk, d), jnp.float32),
            pltpu.VMEM_SHARED((n, d), jnp.float32),
            pltpu.SemaphoreType.DMA,
            pltpu.SemaphoreType.DMA,
        ],
    )
    def k(tab_hbm, src_hbm, dst_hbm, z_hbm, out_hbm,
          src_v, dst_v, rows_v, acc_sh, sem0, sem1):
        c = lax.axis_index("c")
        s = lax.axis_index("s")
        wid = c * _NS + s
        sems = (sem0, sem1)
        pltpu.sync_copy(z_hbm, acc_sh.at[pl.ds(s * rows, rows)])
        plsc.subcore_barrier()

        def chunk(i, b, cnt):
            pltpu.make_async_copy(tab_hbm.at[src_v.at[i]],
                                  rows_v.at[b], sems[b]).wait()
            pltpu.sync_copy(rows_v.at[b], acc_sh.at[dst_v.at[i]], add=True)

            @pl.when(i + 2 < cnt)
            def _():
                pltpu.async_copy(tab_hbm.at[src_v.at[i + 2]],
                                 rows_v.at[b], sems[b])

        for base, cnt in ((0, ph0), (ph0, ph1)):
            pltpu.sync_copy(src_hbm.at[wid, pl.ds(base, cnt)],
                            src_v.at[pl.ds(0, cnt)])
            pltpu.sync_copy(dst_hbm.at[wid, pl.ds(base, cnt)],
                            dst_v.at[pl.ds(0, cnt)])
            pltpu.async_copy(tab_hbm.at[src_v.at[0]], rows_v.at[0], sem0)
            pltpu.async_copy(tab_hbm.at[src_v.at[1]], rows_v.at[1], sem1)

            def body(i2, carry):
                chunk(i2 * 2, 0, cnt)
                chunk(i2 * 2 + 1, 1, cnt)
                return carry

            lax.fori_loop(0, cnt // 2, body, 0)
            if cnt % 2:
                chunk(cnt - 1, 0, cnt)

        plsc.subcore_barrier()
        pltpu.sync_copy(acc_sh.at[pl.ds(s * rows, rows)], out_hbm.at[c, s])

    return k(table, src3, dst3, zeros_blk).reshape(_NC, n, d)


def _dinv(deg_ref):
    deg = deg_ref[0, :, 0:1] + deg_ref[1, :, 0:1] + 1.0
    return lax.rsqrt(deg)


def _tc_scale_mm(x, w, degp):
    def body(x_ref, w_ref, deg_ref, o_ref):
        o_ref[...] = jnp.dot(x_ref[...], w_ref[...],
                             preferred_element_type=jnp.float32) * _dinv(deg_ref)

    return pl.pallas_call(
        body,
        out_shape=jax.ShapeDtypeStruct((x.shape[0], w.shape[1]), jnp.float32),
    )(x, w, degp)


def _tc_mid(accp, g1, degp, b1, gamma, beta, w2):
    n = g1.shape[0]

    def body(acc_ref, g_ref, deg_ref, b_ref, ga_ref, be_ref, w_ref, o_ref):
        dinv = _dinv(deg_ref)
        m = (acc_ref[0] + acc_ref[1] + g_ref[...]) * dinv + b_ref[...]
        mean = jnp.mean(m, axis=0, keepdims=True)
        cen = m - mean
        var = jnp.mean(cen * cen, axis=0, keepdims=True)
        xn = cen * lax.rsqrt(var + _EPS) * ga_ref[...] + be_ref[...]
        xr = jnp.maximum(xn, 0.0)
        o_ref[...] = jnp.dot(xr, w_ref[...],
                             preferred_element_type=jnp.float32) * dinv

    return pl.pallas_call(
        body,
        out_shape=jax.ShapeDtypeStruct((n, w2.shape[1]), jnp.float32),
    )(accp, g1, degp, b1, gamma, beta, w2)


def _tc_final(accp, g2, degp, b2, d_out):
    n = g2.shape[0]

    def body(acc_ref, g_ref, deg_ref, b_ref, o_ref):
        acc = acc_ref[0, :, :d_out] + acc_ref[1, :, :d_out] + g_ref[:, :d_out]
        o_ref[...] = acc * _dinv(deg_ref) + b_ref[...]

    return pl.pallas_call(
        body,
        out_shape=jax.ShapeDtypeStruct((n, d_out), jnp.float32),
    )(accp, g2, degp, b2)


def kernel(x, edge_index, W1, b1, gamma, beta, W2, b2):
    n, d_in = x.shape
    e = edge_index.shape[1]
    d_hid = W1.shape[1]
    d_out = W2.shape[1]
    j = e // (_NW * _K)
    rows = n // _NS

    src3 = edge_index[0].reshape(_NW, j, _K)
    dst3 = edge_index[1].reshape(_NW, j, _K)
    ones_blk = jnp.ones((_K, d_hid), jnp.float32)
    zh = jnp.zeros((rows, d_hid), jnp.float32)

    w2p = jnp.pad(W2, ((0, 0), (0, d_hid - d_out)))

    degp = _sc_degree(dst3, ones_blk, zh, n)[:, :, :16]
    g1 = _tc_scale_mm(x, W1, degp)
    acc1 = _sc_scatter(g1, src3, dst3, zh)
    g2 = _tc_mid(acc1, g1, degp, b1.reshape(1, -1), gamma.reshape(1, -1),
                 beta.reshape(1, -1), w2p)
    acc2 = _sc_scatter(g2, src3, dst3, zh)
    return _tc_final(acc2, g2, degp, b2.reshape(1, -1), d_out)


# speedup vs baseline: 2.6444x; 1.0020x over previous
import functools

import jax
import jax.numpy as jnp
from jax import lax
from jax.experimental import pallas as pl
from jax.experimental.pallas import tpu as pltpu
from jax.experimental.pallas import tpu_sc as plsc

_EPS = 1e-5
_NC = 2
_NS = 16
_NW = _NC * _NS
_K = 80


def _sc_mesh():
    return plsc.VectorSubcoreMesh(core_axis_name="c", subcore_axis_name="s")


def _sc_degree(dst3, ones_blk, zeros_blk, n):
    j = dst3.shape[1]
    d = ones_blk.shape[1]
    rows = n // _NS

    @functools.partial(
        pl.kernel,
        mesh=_sc_mesh(),
        out_type=jax.ShapeDtypeStruct((_NC, _NS, rows, d), jnp.float32),
        scratch_types=[
            pltpu.VMEM((j, _K), jnp.int32),
            pltpu.VMEM((_K, d), jnp.float32),
            pltpu.VMEM_SHARED((n, d), jnp.float32),
            pltpu.SemaphoreType.DMA,
        ],
    )
    def k(dst_hbm, ones_hbm, z_hbm, out_hbm, dst_v, ones_v, acc_sh, sem):
        c = lax.axis_index("c")
        s = lax.axis_index("s")
        wid = c * _NS + s
        pltpu.sync_copy(z_hbm, acc_sh.at[pl.ds(s * rows, rows)])
        pltpu.sync_copy(dst_hbm.at[wid], dst_v)
        pltpu.sync_copy(ones_hbm, ones_v)
        plsc.subcore_barrier()

        def body(i, carry):
            pltpu.async_copy(ones_v, acc_sh.at[dst_v.at[i]], sem, add=True)
            return carry

        lax.fori_loop(0, j, body, 0)

        def drain(i, carry):
            pltpu.make_async_copy(ones_v, acc_sh.at[dst_v.at[i]], sem).wait()
            return carry

        lax.fori_loop(0, j, drain, 0)
        plsc.subcore_barrier()
        pltpu.sync_copy(acc_sh.at[pl.ds(s * rows, rows)], out_hbm.at[c, s])

    return k(dst3, ones_blk, zeros_blk).reshape(_NC, n, d)


def _sc_scatter(table, src3, dst3, zeros_blk):
    n, d = table.shape
    j = src3.shape[1]
    kk = src3.shape[2]
    phb = 32
    rows = n // _NS
    nb = 4

    @functools.partial(
        pl.kernel,
        mesh=_sc_mesh(),
        out_type=jax.ShapeDtypeStruct((_NC, _NS, rows, d), jnp.float32),
        scratch_types=[
            pltpu.VMEM((phb, kk), jnp.int32),
            pltpu.VMEM((phb, kk), jnp.int32),
            pltpu.VMEM((nb, kk, d), jnp.float32),
            pltpu.VMEM_SHARED((n, d), jnp.float32),
            pltpu.SemaphoreType.DMA,
            pltpu.SemaphoreType.DMA,
            pltpu.SemaphoreType.DMA,
            pltpu.SemaphoreType.DMA,
            pltpu.SemaphoreType.DMA,
        ],
    )
    def k(tab_hbm, src_hbm, dst_hbm, z_hbm, out_hbm,
          src_v, dst_v, rows_v, acc_sh, sg0, sg1, sg2, sg3, sem_s):
        c = lax.axis_index("c")
        s = lax.axis_index("s")
        wid = c * _NS + s
        sg = (sg0, sg1, sg2, sg3)
        pltpu.sync_copy(z_hbm, acc_sh.at[pl.ds(s * rows, rows)])
        plsc.subcore_barrier()

        def chunk(i, b, cnt):
            pltpu.make_async_copy(tab_hbm.at[src_v.at[i]],
                                  rows_v.at[b], sg[b]).wait()
            pltpu.async_copy(rows_v.at[b], acc_sh.at[dst_v.at[i]], sem_s,
                             add=True)

            @pl.when(i + 2 < cnt)
            def _():
                b2 = (b + 2) % nb

                @pl.when(i >= 2)
                def _():
                    pltpu.make_async_copy(rows_v.at[b2],
                                          acc_sh.at[dst_v.at[i]],
                                          sem_s).wait()
                pltpu.async_copy(tab_hbm.at[src_v.at[i + 2]],
                                 rows_v.at[b2], sg[b2])

        base = 0
        while base < j:
            cnt = min(phb, j - base)
            pltpu.sync_copy(src_hbm.at[wid, pl.ds(base, cnt)],
                            src_v.at[pl.ds(0, cnt)])
            pltpu.sync_copy(dst_hbm.at[wid, pl.ds(base, cnt)],
                            dst_v.at[pl.ds(0, cnt)])
            pltpu.async_copy(tab_hbm.at[src_v.at[0]], rows_v.at[0], sg0)
            pltpu.async_copy(tab_hbm.at[src_v.at[1]], rows_v.at[1], sg1)

            def body(i4, carry):
                for b in range(nb):
                    i = i4 * nb + b

                    @pl.when(i < cnt)
                    def _():
                        chunk(i, b, cnt)
                return carry

            lax.fori_loop(0, (cnt + nb - 1) // nb, body, 0)

            def drain(i, carry):
                pltpu.make_async_copy(rows_v.at[0], acc_sh.at[dst_v.at[0]],
                                      sem_s).wait()
                return carry

            lax.fori_loop(0, nb, drain, 0)
            base += phb

        plsc.subcore_barrier()
        pltpu.sync_copy(acc_sh.at[pl.ds(s * rows, rows)], out_hbm.at[c, s])

    return k(table, src3, dst3, zeros_blk).reshape(_NC, n, d)


def _dinv(deg_ref):
    deg = deg_ref[0, :, 0:1] + deg_ref[1, :, 0:1] + 1.0
    return lax.rsqrt(deg)


def _tc_scale_mm(x, w, degp):
    def body(x_ref, w_ref, deg_ref, o_ref):
        o_ref[...] = jnp.dot(x_ref[...], w_ref[...],
                             preferred_element_type=jnp.float32) * _dinv(deg_ref)

    return pl.pallas_call(
        body,
        out_shape=jax.ShapeDtypeStruct((x.shape[0], w.shape[1]), jnp.float32),
    )(x, w, degp)


def _tc_mid(accp, g1, degp, b1, gamma, beta, w2):
    n = g1.shape[0]

    def body(acc_ref, g_ref, deg_ref, b_ref, ga_ref, be_ref, w_ref, o_ref):
        dinv = _dinv(deg_ref)
        m = (acc_ref[0] + acc_ref[1] + g_ref[...]) * dinv + b_ref[...]
        mean = jnp.mean(m, axis=0, keepdims=True)
        cen = m - mean
        var = jnp.mean(cen * cen, axis=0, keepdims=True)
        xn = cen * lax.rsqrt(var + _EPS) * ga_ref[...] + be_ref[...]
        xr = jnp.maximum(xn, 0.0)
        o_ref[...] = jnp.dot(xr, w_ref[...],
                             preferred_element_type=jnp.float32) * dinv

    return pl.pallas_call(
        body,
        out_shape=jax.ShapeDtypeStruct((n, w2.shape[1]), jnp.float32),
    )(accp, g1, degp, b1, gamma, beta, w2)


def _tc_final(accp, g2, degp, b2, d_out):
    n = g2.shape[0]

    def body(acc_ref, g_ref, deg_ref, b_ref, o_ref):
        acc = acc_ref[0, :, :d_out] + acc_ref[1, :, :d_out] + g_ref[:, :d_out]
        o_ref[...] = acc * _dinv(deg_ref) + b_ref[...]

    return pl.pallas_call(
        body,
        out_shape=jax.ShapeDtypeStruct((n, d_out), jnp.float32),
    )(accp, g2, degp, b2)


def kernel(x, edge_index, W1, b1, gamma, beta, W2, b2):
    n, d_in = x.shape
    e = edge_index.shape[1]
    d_hid = W1.shape[1]
    d_out = W2.shape[1]
    j = e // (_NW * _K)
    rows = n // _NS

    src3 = edge_index[0].reshape(_NW, j, _K)
    dst3 = edge_index[1].reshape(_NW, j, _K)
    ones_blk = jnp.ones((_K, d_hid), jnp.float32)
    zh = jnp.zeros((rows, d_hid), jnp.float32)

    w2p = jnp.pad(W2, ((0, 0), (0, d_hid - d_out)))

    degp = _sc_degree(dst3, ones_blk, zh, n)[:, :, :16]
    g1 = _tc_scale_mm(x, W1, degp)
    acc1 = _sc_scatter(g1, src3, dst3, zh)
    g2 = _tc_mid(acc1, g1, degp, b1.reshape(1, -1), gamma.reshape(1, -1),
                 beta.reshape(1, -1), w2p)
    acc2 = _sc_scatter(g2, src3, dst3, zh)
    return _tc_final(acc2, g2, degp, b2.reshape(1, -1), d_out)
